# baseline (speedup 1.0000x reference)
import functools

import jax
import jax.numpy as jnp
import numpy as np
from jax import lax
from jax.experimental import pallas as pl
from jax.experimental.pallas import tpu as pltpu
from jax.experimental.pallas import tpu_sc as plsc

N = 10000
NPAD = 10240
E = 320000
D_IN = 128
H = 64
NF = 8

NC = 2
NS = 16
NW = NC * NS

_INTERPRET = False


def _in_mlp_body(x_ref, w_ref, b_ref, o_ref):
    o_ref[...] = jnp.tanh(
        jnp.dot(x_ref[...], w_ref[...], preferred_element_type=jnp.float32)
        + b_ref[...]
    )


def _input_mlp(x, W_in, b_in):
    BN = 2000
    return pl.pallas_call(
        _in_mlp_body,
        grid=(N // BN,),
        in_specs=[
            pl.BlockSpec((BN, D_IN), lambda i: (i, 0)),
            pl.BlockSpec((D_IN, H), lambda i: (0, 0)),
            pl.BlockSpec((1, H), lambda i: (0, 0)),
        ],
        out_specs=pl.BlockSpec((BN, H), lambda i: (i, 0)),
        out_shape=jax.ShapeDtypeStruct((NPAD, H), jnp.float32),
        interpret=_INTERPRET,
    )(x, W_in, b_in.reshape(1, H))


def _gate_body(hp_ref, hn_ref, wgp_ref, wgn_ref, bg_ref, wo_ref, bo_ref, o_ref):
    hp = hp_ref[0] + hp_ref[1]
    hn = hn_ref[0] + hn_ref[1]
    g = jax.nn.sigmoid(
        jnp.dot(hp, wgp_ref[...], preferred_element_type=jnp.float32)
        + jnp.dot(hn, wgn_ref[...], preferred_element_type=jnp.float32)
        + bg_ref[...]
    )
    h = g * hp + (1.0 - g) * hn
    o_ref[...] = jnp.tanh(
        jnp.dot(h, wo_ref[...], preferred_element_type=jnp.float32) + bo_ref[...]
    )


def _gate_update(hp, hn, W_gate, b_gate, W_out, b_out):
    BN = 2048
    return pl.pallas_call(
        _gate_body,
        grid=(NPAD // BN,),
        in_specs=[
            pl.BlockSpec((NC, BN, H), lambda i: (0, i, 0)),
            pl.BlockSpec((NC, BN, H), lambda i: (0, i, 0)),
            pl.BlockSpec((H, H), lambda i: (0, 0)),
            pl.BlockSpec((H, H), lambda i: (0, 0)),
            pl.BlockSpec((1, H), lambda i: (0, 0)),
            pl.BlockSpec((H, H), lambda i: (0, 0)),
            pl.BlockSpec((1, H), lambda i: (0, 0)),
        ],
        out_specs=pl.BlockSpec((BN, H), lambda i: (i, 0)),
        out_shape=jax.ShapeDtypeStruct((NPAD, H), jnp.float32),
        interpret=_INTERPRET,
    )(hp, hn, W_gate[:H], W_gate[H:], b_gate.reshape(1, H), W_out,
      b_out.reshape(1, H))


def _edge_body(hu_ref, hv_ref, t_ref, w1u_ref, w1v_ref, w1a_ref, w1m_ref,
               w1s_ref, w1c_ref, b1_ref, w2_ref, b2_ref, o_ref):
    hu = hu_ref[...]
    hv = hv_ref[...]
    dot = functools.partial(jnp.dot, preferred_element_type=jnp.float32)
    acc = dot(hu, w1u_ref[...]) + dot(hv, w1v_ref[...])
    acc += dot(jnp.abs(hu - hv), w1a_ref[...]) + dot(hu * hv, w1m_ref[...])
    t = jnp.clip(t_ref[...], 0.0, 1.0)
    freqs = np.pi * (jnp.arange(1, NF + 1, dtype=jnp.int32)
                     .astype(jnp.float32)).reshape(1, NF)
    ang = t * freqs
    acc += dot(jnp.sin(ang), w1s_ref[...]) + dot(jnp.cos(ang), w1c_ref[...])
    hid = jax.nn.relu(acc + b1_ref[...])
    o_ref[...] = dot(hid, w2_ref[...]) + b2_ref[...]


def _edge_mlp(hu, hv, ts, W_e1, b_e1, W_e2, b_e2):
    BE = 4000
    out = pl.pallas_call(
        _edge_body,
        grid=(E // BE,),
        in_specs=[
            pl.BlockSpec((BE, H), lambda i: (i, 0)),
            pl.BlockSpec((BE, H), lambda i: (i, 0)),
            pl.BlockSpec((BE, 1), lambda i: (i, 0)),
            pl.BlockSpec((H, H), lambda i: (0, 0)),
            pl.BlockSpec((H, H), lambda i: (0, 0)),
            pl.BlockSpec((H, H), lambda i: (0, 0)),
            pl.BlockSpec((H, H), lambda i: (0, 0)),
            pl.BlockSpec((NF, H), lambda i: (0, 0)),
            pl.BlockSpec((NF, H), lambda i: (0, 0)),
            pl.BlockSpec((1, H), lambda i: (0, 0)),
            pl.BlockSpec((H, 1), lambda i: (0, 0)),
            pl.BlockSpec((1, 1), lambda i: (0, 0)),
        ],
        out_specs=pl.BlockSpec((BE, 1), lambda i: (i, 0)),
        out_shape=jax.ShapeDtypeStruct((E, 1), jnp.float32),
        interpret=_INTERPRET,
    )(hu, hv, ts.reshape(E, 1), W_e1[:H], W_e1[H:2 * H], W_e1[2 * H:3 * H],
      W_e1[3 * H:4 * H], W_e1[4 * H:4 * H + NF], W_e1[4 * H + NF:],
      b_e1.reshape(1, H), W_e2, b_e2.reshape(1, 1))
    return out[:, 0]


UNIT = 128
UPT = (-(-E // (UNIT * NW)) + 7) // 8 * 8
UNITS = UPT * NW
EPAD = UNITS * UNIT
RPT = NPAD // NS

_sc_mesh = plsc.VectorSubcoreMesh(core_axis_name="c", subcore_axis_name="s",
                                  num_cores=NC, num_subcores=NS)


def _seg_body(h_hbm, prow, pcol, pval, nrow, ncol, nval, zeros_hbm,
              hp_out, hn_out,
              rowbuf, colbuf, valbuf, rows_a, rows_b, h_sh, acc,
              sem_ga, sem_gb, sem_sa, sem_sb):
    c = lax.axis_index("c")
    s = lax.axis_index("s")
    wid = c * NS + s

    pltpu.sync_copy(h_hbm.at[pl.ds(s * RPT, RPT)], h_sh.at[pl.ds(s * RPT, RPT)])
    pltpu.sync_copy(zeros_hbm, acc.at[pl.ds(s * RPT, RPT)])
    plsc.subcore_barrier()

    def scale(rows_v, j):
        def scale_group(g, _):
            vv = valbuf[j, pl.ds(g * 16, 16)]
            for l in range(16):
                e = g * 16 + l
                v = vv[l]
                for k in range(H // 16):
                    sl = rows_v[e, pl.ds(k * 16, 16)]
                    rows_v[e, pl.ds(k * 16, 16)] = sl * v
            return 0

        lax.fori_loop(0, UNIT // 16, scale_group, 0, unroll=2)

    def run_adj(row2d, col2d, val2d):
        base_u = wid * UPT
        pltpu.sync_copy(row2d.at[pl.ds(base_u, UPT)], rowbuf)
        pltpu.sync_copy(col2d.at[pl.ds(base_u, UPT)], colbuf)
        pltpu.sync_copy(val2d.at[pl.ds(base_u, UPT)], valbuf)

        def gather(j, rows_v, g_sem):
            pltpu.async_copy(h_sh.at[colbuf.at[j]], rows_v, g_sem)

        def wait_gather(rows_v, g_sem):
            pltpu.make_async_copy(h_sh.at[colbuf.at[0]], rows_v, g_sem).wait()

        gather(0, rows_a, sem_ga)
        gather(1, rows_b, sem_gb)

        def pair_step(t, _):
            j0 = 2 * t
            wait_gather(rows_a, sem_ga)
            scale(rows_a, j0)
            sc_a = pltpu.async_copy(rows_a, acc.at[rowbuf.at[j0]],
                                    sem_sa, add=True)
            wait_gather(rows_b, sem_gb)
            scale(rows_b, j0 + 1)
            sc_b = pltpu.async_copy(rows_b, acc.at[rowbuf.at[j0 + 1]],
                                    sem_sb, add=True)

            @pl.when(t < UPT // 2 - 1)
            def _prefetch():
                sc_a.wait()
                gather(j0 + 2, rows_a, sem_ga)
                sc_b.wait()
                gather(j0 + 3, rows_b, sem_gb)

            return 0

        lax.fori_loop(0, UPT // 2, pair_step, 0)
        pltpu.make_async_copy(rows_a, acc.at[rowbuf.at[0]], sem_sa).wait()
        pltpu.make_async_copy(rows_b, acc.at[rowbuf.at[0]], sem_sb).wait()

    run_adj(prow, pcol, pval)
    plsc.subcore_barrier()
    pltpu.sync_copy(acc.at[pl.ds(s * RPT, RPT)],
                    hp_out.at[c, pl.ds(s * RPT, RPT)])
    pltpu.sync_copy(zeros_hbm, acc.at[pl.ds(s * RPT, RPT)])
    plsc.subcore_barrier()
    run_adj(nrow, ncol, nval)
    plsc.subcore_barrier()
    pltpu.sync_copy(acc.at[pl.ds(s * RPT, RPT)],
                    hn_out.at[c, pl.ds(s * RPT, RPT)])


_seg_kernel = functools.partial(
    pl.kernel,
    _seg_body,
    out_type=[jax.ShapeDtypeStruct((NC, NPAD, H), jnp.float32),
              jax.ShapeDtypeStruct((NC, NPAD, H), jnp.float32)],
    mesh=_sc_mesh,
    compiler_params=pltpu.CompilerParams(use_tc_tiling_on_sc=False),
    scratch_types=[
        pltpu.VMEM((UPT, UNIT), jnp.int32),
        pltpu.VMEM((UPT, UNIT), jnp.int32),
        pltpu.VMEM((UPT, UNIT), jnp.float32),
        pltpu.VMEM((UNIT, H), jnp.float32),
        pltpu.VMEM((UNIT, H), jnp.float32),
        pltpu.VMEM_SHARED((NPAD, H), jnp.float32),
        pltpu.VMEM_SHARED((NPAD, H), jnp.float32),
        pltpu.SemaphoreType.DMA,
        pltpu.SemaphoreType.DMA,
        pltpu.SemaphoreType.DMA,
        pltpu.SemaphoreType.DMA,
    ],
)()


def _pad_edges(row, col, val):
    pad = EPAD - E
    row = jnp.concatenate([row, jnp.zeros((pad,), row.dtype)]).reshape(UNITS, UNIT)
    col = jnp.concatenate([col, jnp.zeros((pad,), col.dtype)]).reshape(UNITS, UNIT)
    val = jnp.concatenate([val, jnp.zeros((pad,), val.dtype)]).reshape(UNITS, UNIT)
    return row, col, val


def _segment_sums(h, pos_e, neg_e, zeros):
    hp2, hn2 = _seg_kernel(h, *pos_e, *neg_e, zeros)
    return hp2, hn2


def _gather_body(h_hbm, ubuf2d, vbuf2d, hu_out, hv_out, idxbuf, rows_a, rows_b,
                 h_sh, sem_ga, sem_gb, sem_sa, sem_sb):
    c = lax.axis_index("c")
    s = lax.axis_index("s")
    wid = c * NS + s
    base_u = wid * UPT

    pltpu.sync_copy(h_hbm.at[pl.ds(s * RPT, RPT)], h_sh.at[pl.ds(s * RPT, RPT)])
    plsc.subcore_barrier()

    def run_side(idx2d, out):
        pltpu.sync_copy(idx2d.at[pl.ds(base_u, UPT)], idxbuf)

        def gather(j, rows_v, g_sem):
            pltpu.async_copy(h_sh.at[idxbuf.at[j]], rows_v, g_sem)

        def wait_gather(rows_v, g_sem):
            pltpu.make_async_copy(h_sh.at[idxbuf.at[0]], rows_v, g_sem).wait()

        gather(0, rows_a, sem_ga)
        gather(1, rows_b, sem_gb)

        def pair_step(t, _):
            j0 = 2 * t
            wait_gather(rows_a, sem_ga)
            sc_a = pltpu.async_copy(
                rows_a, out.at[pl.ds((base_u + j0) * UNIT, UNIT)], sem_sa)
            wait_gather(rows_b, sem_gb)
            sc_b = pltpu.async_copy(
                rows_b, out.at[pl.ds((base_u + j0 + 1) * UNIT, UNIT)], sem_sb)

            @pl.when(t < UPT // 2 - 1)
            def _prefetch():
                sc_a.wait()
                gather(j0 + 2, rows_a, sem_ga)
                sc_b.wait()
                gather(j0 + 3, rows_b, sem_gb)

            return 0

        lax.fori_loop(0, UPT // 2, pair_step, 0)
        pltpu.make_async_copy(rows_a, out.at[pl.ds(0, UNIT)], sem_sa).wait()
        pltpu.make_async_copy(rows_b, out.at[pl.ds(0, UNIT)], sem_sb).wait()

    run_side(ubuf2d, hu_out)
    run_side(vbuf2d, hv_out)


_gather_kernel = functools.partial(
    pl.kernel,
    _gather_body,
    out_type=[jax.ShapeDtypeStruct((EPAD, H), jnp.float32),
              jax.ShapeDtypeStruct((EPAD, H), jnp.float32)],
    mesh=_sc_mesh,
    scratch_types=[
        pltpu.VMEM((UPT, UNIT), jnp.int32),
        pltpu.VMEM((UNIT, H), jnp.float32),
        pltpu.VMEM((UNIT, H), jnp.float32),
        pltpu.VMEM_SHARED((NPAD, H), jnp.float32),
        pltpu.SemaphoreType.DMA,
        pltpu.SemaphoreType.DMA,
        pltpu.SemaphoreType.DMA,
        pltpu.SemaphoreType.DMA,
    ],
    compiler_params=pltpu.CompilerParams(use_tc_tiling_on_sc=False),
)()


def _edge_gather(h, u2d, v2d):
    return _gather_kernel(h, u2d, v2d)


def kernel(x, A_pos_idx, A_pos_val, A_neg_idx, A_neg_val, edge_index,
           edge_timestamp, W_in, b_in, W_gate0, b_gate0, W_gate1, b_gate1,
           W_out0, b_out0, W_out1, b_out1, W_e1, b_e1, W_e2, b_e2):
    h = _input_mlp(x, W_in, b_in)
    pos_e = _pad_edges(A_pos_idx[0], A_pos_idx[1], A_pos_val)
    neg_e = _pad_edges(A_neg_idx[0], A_neg_idx[1], A_neg_val)
    zeros = jnp.zeros((RPT, H), jnp.float32)
    pad_i = jnp.zeros((EPAD - E,), edge_index.dtype)
    u2d = jnp.concatenate([edge_index[0], pad_i]).reshape(UNITS, UNIT)
    v2d = jnp.concatenate([edge_index[1], pad_i]).reshape(UNITS, UNIT)
    Wg = [(W_gate0, b_gate0), (W_gate1, b_gate1)]
    Wo = [(W_out0, b_out0), (W_out1, b_out1)]
    for hop in range(2):
        hp2, hn2 = _segment_sums(h, pos_e, neg_e, zeros)
        h = _gate_update(hp2, hn2, Wg[hop][0], Wg[hop][1], Wo[hop][0], Wo[hop][1])
    hu, hv = _edge_gather(h, u2d, v2d)
    return _edge_mlp(hu, hv, edge_timestamp, W_e1, b_e1, W_e2, b_e2)


# speedup vs baseline: 1.2214x; 1.2214x over previous
import functools

import jax
import jax.numpy as jnp
import numpy as np
from jax import lax
from jax.experimental import pallas as pl
from jax.experimental.pallas import tpu as pltpu
from jax.experimental.pallas import tpu_sc as plsc

N = 10000
NPAD = 10240
E = 320000
D_IN = 128
H = 64
H2 = 2 * H
NF = 8
N2 = N // 2
NP2 = NPAD // 2
E2 = E // 2

NC = 2
NS = 16
NW = NC * NS

_INTERPRET = False


def _kron2(w):
    z = jnp.zeros_like(w)
    return jnp.concatenate(
        [jnp.concatenate([w, z], axis=1), jnp.concatenate([z, w], axis=1)],
        axis=0)


def _tile2(b):
    return jnp.concatenate([b, b]).reshape(1, -1)


def _in_mlp_body(x_ref, w_ref, b_ref, o_ref):
    o_ref[...] = jnp.tanh(
        jnp.dot(x_ref[...], w_ref[...], preferred_element_type=jnp.float32)
        + b_ref[...]
    )


def _input_mlp(x, W_in, b_in):
    BN = 2000
    return pl.pallas_call(
        _in_mlp_body,
        grid=(N // BN,),
        in_specs=[
            pl.BlockSpec((BN, D_IN), lambda i: (i, 0)),
            pl.BlockSpec((D_IN, H), lambda i: (0, 0)),
            pl.BlockSpec((1, H), lambda i: (0, 0)),
        ],
        out_specs=pl.BlockSpec((BN, H), lambda i: (i, 0)),
        out_shape=jax.ShapeDtypeStruct((NPAD, H), jnp.float32),
        interpret=_INTERPRET,
    )(x, W_in, b_in.reshape(1, H))


def _gate_body(hp_ref, hn_ref, wgp_ref, wgn_ref, bg_ref, wo_ref, bo_ref, o_ref):
    hp = hp_ref[0] + hp_ref[1]
    hn = hn_ref[0] + hn_ref[1]
    g = jax.nn.sigmoid(
        jnp.dot(hp, wgp_ref[...], preferred_element_type=jnp.float32)
        + jnp.dot(hn, wgn_ref[...], preferred_element_type=jnp.float32)
        + bg_ref[...]
    )
    h = g * hp + (1.0 - g) * hn
    o_ref[...] = jnp.tanh(
        jnp.dot(h, wo_ref[...], preferred_element_type=jnp.float32) + bo_ref[...]
    )


def _gate_update(hp2, hn2, W_gate, b_gate, W_out, b_out):
    BN = 2048
    return pl.pallas_call(
        _gate_body,
        grid=(NPAD // BN,),
        in_specs=[
            pl.BlockSpec((NC, BN, H), lambda i: (0, i, 0)),
            pl.BlockSpec((NC, BN, H), lambda i: (0, i, 0)),
            pl.BlockSpec((H, H), lambda i: (0, 0)),
            pl.BlockSpec((H, H), lambda i: (0, 0)),
            pl.BlockSpec((1, H), lambda i: (0, 0)),
            pl.BlockSpec((H, H), lambda i: (0, 0)),
            pl.BlockSpec((1, H), lambda i: (0, 0)),
        ],
        out_specs=pl.BlockSpec((BN, H), lambda i: (i, 0)),
        out_shape=jax.ShapeDtypeStruct((NPAD, H), jnp.float32),
        interpret=_INTERPRET,
    )(hp2, hn2, W_gate[:H], W_gate[H:], b_gate.reshape(1, H), W_out,
      b_out.reshape(1, H))


def _edge_body(hu_ref, hv_ref, t_ref, w1u_ref, w1v_ref, w1a_ref, w1m_ref,
               w1t_ref, b1_ref, w2_ref, b2_ref, o_ref):
    hu = hu_ref[...]
    hv = hv_ref[...]
    dot = functools.partial(jnp.dot, preferred_element_type=jnp.float32)
    acc = dot(hu, w1u_ref[...]) + dot(hv, w1v_ref[...])
    acc += dot(jnp.abs(hu - hv), w1a_ref[...]) + dot(hu * hv, w1m_ref[...])
    t = jnp.clip(t_ref[...], 0.0, 1.0)
    freqs = np.pi * (jnp.arange(1, NF + 1, dtype=jnp.int32)
                     .astype(jnp.float32)).reshape(1, NF)
    ang_e = t[:, 0:1] * freqs
    ang_o = t[:, 1:2] * freqs
    tf = jnp.concatenate(
        [jnp.sin(ang_e), jnp.cos(ang_e), jnp.sin(ang_o), jnp.cos(ang_o)],
        axis=1)
    acc += dot(tf, w1t_ref[...])
    hid = jax.nn.relu(acc + b1_ref[...])
    o_ref[...] = dot(hid, w2_ref[...]) + b2_ref[...]


def _edge_mlp(hu2, hv2, ts, W_e1, b_e1, W_e2, b_e2):
    B2 = 2000
    out = pl.pallas_call(
        _edge_body,
        grid=(E2 // B2,),
        in_specs=[
            pl.BlockSpec((B2, H2), lambda i: (i, 0)),
            pl.BlockSpec((B2, H2), lambda i: (i, 0)),
            pl.BlockSpec((B2, 2), lambda i: (i, 0)),
            pl.BlockSpec((H2, H2), lambda i: (0, 0)),
            pl.BlockSpec((H2, H2), lambda i: (0, 0)),
            pl.BlockSpec((H2, H2), lambda i: (0, 0)),
            pl.BlockSpec((H2, H2), lambda i: (0, 0)),
            pl.BlockSpec((4 * NF, H2), lambda i: (0, 0)),
            pl.BlockSpec((1, H2), lambda i: (0, 0)),
            pl.BlockSpec((H2, 2), lambda i: (0, 0)),
            pl.BlockSpec((1, 2), lambda i: (0, 0)),
        ],
        out_specs=pl.BlockSpec((B2, 2), lambda i: (i, 0)),
        out_shape=jax.ShapeDtypeStruct((E2, 2), jnp.float32),
        interpret=_INTERPRET,
    )(hu2, hv2, ts.reshape(E2, 2), _kron2(W_e1[:H]), _kron2(W_e1[H:2 * H]),
      _kron2(W_e1[2 * H:3 * H]), _kron2(W_e1[3 * H:4 * H]),
      _kron2(W_e1[4 * H:]), _tile2(b_e1), _kron2(W_e2), _tile2(b_e2))
    return out.reshape(E)


UNIT = 128
UH = UNIT // 2
UPT = (-(-E // (UNIT * NW)) + 7) // 8 * 8
UNITS = UPT * NW
EPAD = UNITS * UNIT
RPT = NPAD // NS
RPT2 = RPT // 2

_sc_mesh = plsc.VectorSubcoreMesh(core_axis_name="c", subcore_axis_name="s",
                                  num_cores=NC, num_subcores=NS)


def _seg_body(h_hbm, prow, pcol, pval, nrow, ncol, nval, zeros_hbm,
              hp_out, hn_out,
              rowbuf, colbuf, valbuf, rows_a, rows_b, h_sh, acc,
              sem_ga, sem_gb, sem_sa, sem_sb):
    c = lax.axis_index("c")
    s = lax.axis_index("s")
    wid = c * NS + s

    pltpu.sync_copy(h_hbm.at[pl.ds(s * RPT, RPT)],
                    h_sh.at[pl.ds(s * RPT, RPT)])
    pltpu.sync_copy(zeros_hbm, acc.at[pl.ds(s * RPT, RPT)])
    plsc.subcore_barrier()

    def scale(rows_v, j):
        def scale_group(g, _):
            vv = valbuf[j, pl.ds(g * 16, 16)]
            for l in range(16):
                e = g * 16 + l
                v = vv[l]
                for k in range(H // 16):
                    sl = rows_v[e, pl.ds(k * 16, 16)]
                    rows_v[e, pl.ds(k * 16, 16)] = sl * v
            return 0

        lax.fori_loop(0, UNIT // 16, scale_group, 0, unroll=2)

    def run_adj(row2d, col2d, val2d):
        base_u = wid * UPT
        pltpu.sync_copy(row2d.at[pl.ds(base_u, UPT)], rowbuf)
        pltpu.sync_copy(col2d.at[pl.ds(base_u, UPT)], colbuf)
        pltpu.sync_copy(val2d.at[pl.ds(base_u, UPT)], valbuf)

        def gather(j, rows_v, g_sem):
            pltpu.async_copy(h_sh.at[colbuf.at[j]], rows_v, g_sem)

        def wait_gather(rows_v, g_sem):
            pltpu.make_async_copy(h_sh.at[colbuf.at[0]], rows_v, g_sem).wait()

        gather(0, rows_a, sem_ga)
        gather(1, rows_b, sem_gb)

        def pair_step(t, _):
            j0 = 2 * t
            wait_gather(rows_a, sem_ga)
            scale(rows_a, j0)
            sc_a = pltpu.async_copy(rows_a, acc.at[rowbuf.at[j0]],
                                    sem_sa, add=True)
            wait_gather(rows_b, sem_gb)
            scale(rows_b, j0 + 1)
            sc_b = pltpu.async_copy(rows_b, acc.at[rowbuf.at[j0 + 1]],
                                    sem_sb, add=True)

            @pl.when(t < UPT // 2 - 1)
            def _prefetch():
                sc_a.wait()
                gather(j0 + 2, rows_a, sem_ga)
                sc_b.wait()
                gather(j0 + 3, rows_b, sem_gb)

            return 0

        lax.fori_loop(0, UPT // 2, pair_step, 0)
        pltpu.make_async_copy(rows_a, acc.at[rowbuf.at[0]], sem_sa).wait()
        pltpu.make_async_copy(rows_b, acc.at[rowbuf.at[0]], sem_sb).wait()

    run_adj(prow, pcol, pval)
    plsc.subcore_barrier()
    pltpu.sync_copy(acc.at[pl.ds(s * RPT, RPT)],
                    hp_out.at[c, pl.ds(s * RPT, RPT)])
    pltpu.sync_copy(zeros_hbm, acc.at[pl.ds(s * RPT, RPT)])
    plsc.subcore_barrier()
    run_adj(nrow, ncol, nval)
    plsc.subcore_barrier()
    pltpu.sync_copy(acc.at[pl.ds(s * RPT, RPT)],
                    hn_out.at[c, pl.ds(s * RPT, RPT)])


_seg_kernel = functools.partial(
    pl.kernel,
    _seg_body,
    out_type=[jax.ShapeDtypeStruct((NC, NPAD, H), jnp.float32),
              jax.ShapeDtypeStruct((NC, NPAD, H), jnp.float32)],
    mesh=_sc_mesh,
    compiler_params=pltpu.CompilerParams(use_tc_tiling_on_sc=False),
    scratch_types=[
        pltpu.VMEM((UPT, UNIT), jnp.int32),
        pltpu.VMEM((UPT, UNIT), jnp.int32),
        pltpu.VMEM((UPT, UNIT), jnp.float32),
        pltpu.VMEM((UNIT, H), jnp.float32),
        pltpu.VMEM((UNIT, H), jnp.float32),
        pltpu.VMEM_SHARED((NPAD, H), jnp.float32),
        pltpu.VMEM_SHARED((NPAD, H), jnp.float32),
        pltpu.SemaphoreType.DMA,
        pltpu.SemaphoreType.DMA,
        pltpu.SemaphoreType.DMA,
        pltpu.SemaphoreType.DMA,
    ],
)()


def _pad_edges(row, col, val):
    pad = EPAD - E
    row = jnp.concatenate([row, jnp.zeros((pad,), row.dtype)]).reshape(UNITS, UNIT)
    col = jnp.concatenate([col, jnp.zeros((pad,), col.dtype)]).reshape(UNITS, UNIT)
    val = jnp.concatenate([val, jnp.zeros((pad,), val.dtype)]).reshape(UNITS, UNIT)
    return row, col, val


def _segment_sums(h, pos_e, neg_e, zeros):
    return _seg_kernel(h, *pos_e, *neg_e, zeros)


def _gather_body(h_hbm, ubuf2d, vbuf2d, hu_out, hv_out, idxbuf, rows_a, rows_b,
                 h_sh, sem_ga, sem_gb, sem_sa, sem_sb):
    c = lax.axis_index("c")
    s = lax.axis_index("s")
    wid = c * NS + s
    base_u = wid * UPT

    pltpu.sync_copy(h_hbm.at[pl.ds(s * RPT, RPT)],
                    h_sh.at[pl.ds(s * RPT, RPT)])
    plsc.subcore_barrier()

    def run_side(idx2d, out):
        pltpu.sync_copy(idx2d.at[pl.ds(base_u, UPT)], idxbuf)

        def gather(j, rows_v, g_sem):
            pltpu.async_copy(h_sh.at[idxbuf.at[j]], rows_v, g_sem)

        def wait_gather(rows_v, g_sem):
            pltpu.make_async_copy(h_sh.at[idxbuf.at[0]], rows_v, g_sem).wait()

        gather(0, rows_a, sem_ga)
        gather(1, rows_b, sem_gb)

        def pair_step(t, _):
            j0 = 2 * t
            wait_gather(rows_a, sem_ga)
            sc_a = pltpu.async_copy(
                rows_a, out.at[pl.ds((base_u + j0) * UNIT, UNIT)], sem_sa)
            wait_gather(rows_b, sem_gb)
            sc_b = pltpu.async_copy(
                rows_b, out.at[pl.ds((base_u + j0 + 1) * UNIT, UNIT)], sem_sb)

            @pl.when(t < UPT // 2 - 1)
            def _prefetch():
                sc_a.wait()
                gather(j0 + 2, rows_a, sem_ga)
                sc_b.wait()
                gather(j0 + 3, rows_b, sem_gb)

            return 0

        lax.fori_loop(0, UPT // 2, pair_step, 0)
        pltpu.make_async_copy(rows_a, out.at[pl.ds(0, UNIT)], sem_sa).wait()
        pltpu.make_async_copy(rows_b, out.at[pl.ds(0, UNIT)], sem_sb).wait()

    run_side(ubuf2d, hu_out)
    run_side(vbuf2d, hv_out)


_gather_kernel = functools.partial(
    pl.kernel,
    _gather_body,
    out_type=[jax.ShapeDtypeStruct((EPAD, H), jnp.float32),
              jax.ShapeDtypeStruct((EPAD, H), jnp.float32)],
    mesh=_sc_mesh,
    scratch_types=[
        pltpu.VMEM((UPT, UNIT), jnp.int32),
        pltpu.VMEM((UNIT, H), jnp.float32),
        pltpu.VMEM((UNIT, H), jnp.float32),
        pltpu.VMEM_SHARED((NPAD, H), jnp.float32),
        pltpu.SemaphoreType.DMA,
        pltpu.SemaphoreType.DMA,
        pltpu.SemaphoreType.DMA,
        pltpu.SemaphoreType.DMA,
    ],
    compiler_params=pltpu.CompilerParams(use_tc_tiling_on_sc=False),
)()


def _edge_gather(h, u2d, v2d):
    hu, hv = _gather_kernel(h, u2d, v2d)
    return hu.reshape(EPAD // 2, H2), hv.reshape(EPAD // 2, H2)


def kernel(x, A_pos_idx, A_pos_val, A_neg_idx, A_neg_val, edge_index,
           edge_timestamp, W_in, b_in, W_gate0, b_gate0, W_gate1, b_gate1,
           W_out0, b_out0, W_out1, b_out1, W_e1, b_e1, W_e2, b_e2):
    h = _input_mlp(x, W_in, b_in)
    pos_e = _pad_edges(A_pos_idx[0], A_pos_idx[1], A_pos_val)
    neg_e = _pad_edges(A_neg_idx[0], A_neg_idx[1], A_neg_val)
    zeros = jnp.zeros((RPT, H), jnp.float32)
    pad_i = jnp.zeros((EPAD - E,), edge_index.dtype)
    u2d = jnp.concatenate([edge_index[0], pad_i]).reshape(UNITS, UNIT)
    v2d = jnp.concatenate([edge_index[1], pad_i]).reshape(UNITS, UNIT)
    Wg = [(W_gate0, b_gate0), (W_gate1, b_gate1)]
    Wo = [(W_out0, b_out0), (W_out1, b_out1)]
    for hop in range(2):
        hp2, hn2 = _segment_sums(h, pos_e, neg_e, zeros)
        h = _gate_update(hp2, hn2, Wg[hop][0], Wg[hop][1], Wo[hop][0], Wo[hop][1])
    hu2, hv2 = _edge_gather(h, u2d, v2d)
    return _edge_mlp(hu2, hv2, edge_timestamp, W_e1, b_e1, W_e2, b_e2)
